# baseline (speedup 1.0000x reference)
import jax
import jax.numpy as jnp
from jax import lax
from jax.experimental import pallas as pl
from jax.experimental.pallas import tpu as pltpu
from jax.experimental.pallas import tpu_sc as plsc

_NU, _NI, _NB = 20000, 30000, 20000
_N = _NU + _NI
_NPAD = 51200
_NODES_PER_TILE = _NPAD // 16
_E = 800000
_NSL = 8
_SLD = 16
_CHUNK = 512
_IDXR = _CHUNK // 128
_CHUNKS_PER_TILE = 98
_EDGES_PER_TILE = _CHUNK * _CHUNKS_PER_TILE
_EPAD = 16 * _EDGES_PER_TILE
_ROWS2D = _EPAD // 128
_ROWS_PER_TILE = _ROWS2D // 16
_ZROWS = 128

_mesh = plsc.VectorSubcoreMesh(core_axis_name="c", subcore_axis_name="s")
_cparams = pltpu.CompilerParams(use_tc_tiling_on_sc=False)


def _deg_body(r1, r2, d1, d2, dacc, idx, ones, zeros1d, sem):
    core = lax.axis_index("c")
    sub = lax.axis_index("s")
    node_lo = sub * _NODES_PER_TILE
    row_base = sub * _ROWS_PER_TILE

    def fill_ones(i, _):
        ones[pl.ds(i * 16, 16)] = jnp.full((16,), 1.0, jnp.float32)
        return _
    lax.fori_loop(0, 8, fill_ones, 0)

    def zfill(i, _):
        zeros1d[pl.ds(i * 16, 16)] = jnp.zeros((16,), jnp.float32)
        return _
    lax.fori_loop(0, _NODES_PER_TILE // 16, zfill, 0)

    for g, (rref, dref) in enumerate(((r1, d1), (r2, d2))):
        @pl.when(core == g)
        def _():
            pltpu.sync_copy(zeros1d, dacc.at[pl.ds(node_lo, _NODES_PER_TILE)])
            plsc.subcore_barrier()

            def chunk(g_, _):
                roff = row_base + g_ * 8
                pltpu.sync_copy(rref.at[pl.ds(roff, 8)], idx)
                for j in range(8):
                    pltpu.sync_copy(ones.at[pl.ds(0, 128)],
                                    dacc.at[idx.at[j]], add=True)
                return _
            lax.fori_loop(0, _ROWS_PER_TILE // 8, chunk, 0)
            plsc.subcore_barrier()
            pltpu.sync_copy(dacc.at[pl.ds(node_lo, _NODES_PER_TILE)],
                            dref.at[pl.ds(node_lo, _NODES_PER_TILE)])


_deg_kernel = pl.kernel(
    _deg_body,
    out_type=[jax.ShapeDtypeStruct((_NPAD,), jnp.float32)] * 2,
    mesh=_mesh,
    scratch_types=[
        pltpu.VMEM_SHARED((_NPAD,), jnp.float32),
        pltpu.VMEM((8, 128), jnp.int32),
        pltpu.VMEM((128,), jnp.float32),
        pltpu.VMEM((_NODES_PER_TILE,), jnp.float32),
        pltpu.SemaphoreType.DMA,
    ],
    compiler_params=_cparams,
)


def _prop_body(z0, z1, z2, z3, z4, z5, z6, z7, row2d, col2d,
               o0, o1, o2, o3, o4, o5, o6, o7,
               zspm, acc, idx_row, idx_col, rows, zbuf, isem, gsem, ssem):
    core = lax.axis_index("c")
    sub = lax.axis_index("s")
    node_lo = sub * _NODES_PER_TILE
    row_base = sub * _ROWS_PER_TILE

    def zb(i, _):
        zbuf[i, pl.ds(0, 16)] = jnp.zeros((16,), jnp.float32)
        return _
    lax.fori_loop(0, _ZROWS, zb, 0)

    zo = ((z0, o0), (z1, o1), (z2, o2), (z3, o3),
          (z4, o4), (z5, o5), (z6, o6), (z7, o7))
    for s, (zref, oref) in enumerate(zo):
        @pl.when(core == s // 4)
        def _():
            pltpu.sync_copy(zref.at[pl.ds(node_lo, _NODES_PER_TILE)],
                            zspm.at[pl.ds(node_lo, _NODES_PER_TILE)])

            def zcopy(i, _):
                pltpu.sync_copy(zbuf, acc.at[pl.ds(node_lo + i * _ZROWS, _ZROWS)])
                return _
            lax.fori_loop(0, _NODES_PER_TILE // _ZROWS, zcopy, 0)
            plsc.subcore_barrier()

            def idx_cp(g, b):
                roff = row_base + g * _IDXR
                return (pltpu.make_async_copy(row2d.at[pl.ds(roff, _IDXR)],
                                              idx_row.at[b], isem),
                        pltpu.make_async_copy(col2d.at[pl.ds(roff, _IDXR)],
                                              idx_col.at[b], isem))

            def gathers(b):
                return [pltpu.make_async_copy(
                            zspm.at[idx_col.at[b].at[j]],
                            rows.at[b].at[pl.ds(j * 128, 128)], gsem)
                        for j in range(_IDXR)]

            def scat_chunk(b):
                descs = [pltpu.async_copy(
                             rows.at[b].at[pl.ds(j * 128, 128)],
                             acc.at[idx_row.at[b].at[j]], ssem, add=True)
                         for j in range(_IDXR)]
                for d in descs:
                    d.wait()

            for d in idx_cp(0, 0):
                d.start()
            for d in idx_cp(0, 0):
                d.wait()
            for d in gathers(0):
                d.start()
            for d in idx_cp(1, 1):
                d.start()

            def pair(gi, _):
                for b in range(2):
                    g = 2 * gi + b
                    for d in gathers(b):
                        d.wait()
                    for d in idx_cp(g + 1, 1 - b):
                        d.wait()
                    for d in gathers(1 - b):
                        d.start()
                    scat_chunk(b)
                    for d in idx_cp(g + 2, b):
                        d.start()
                return _
            lax.fori_loop(0, (_CHUNKS_PER_TILE - 2) // 2, pair, 0)

            gl = _CHUNKS_PER_TILE - 2
            for d in gathers(0):
                d.wait()
            for d in idx_cp(gl + 1, 1):
                d.wait()
            for d in gathers(1):
                d.start()
            scat_chunk(0)
            for d in gathers(1):
                d.wait()
            scat_chunk(1)

            plsc.subcore_barrier()
            pltpu.sync_copy(acc.at[pl.ds(node_lo, _NODES_PER_TILE)],
                            oref.at[pl.ds(node_lo, _NODES_PER_TILE)])
            plsc.subcore_barrier()


_prop_kernel = pl.kernel(
    _prop_body,
    out_type=[jax.ShapeDtypeStruct((_NPAD, _SLD), jnp.float32)] * _NSL,
    mesh=_mesh,
    scratch_types=[
        pltpu.VMEM_SHARED((_NPAD, _SLD), jnp.float32),
        pltpu.VMEM_SHARED((_NPAD, _SLD), jnp.float32),
        pltpu.VMEM((2, _IDXR, 128), jnp.int32),
        pltpu.VMEM((2, _IDXR, 128), jnp.int32),
        pltpu.VMEM((2, _CHUNK, _SLD), jnp.float32),
        pltpu.VMEM((_ZROWS, _SLD), jnp.float32),
        pltpu.SemaphoreType.DMA,
        pltpu.SemaphoreType.DMA,
        pltpu.SemaphoreType.DMA,
    ],
    compiler_params=_cparams,
)


def _pad_edges(row, col):
    pad = _EPAD - _E
    padidx = _N + (jnp.arange(pad, dtype=jnp.int32) % 16)
    row_p = jnp.concatenate([row, padidx]).reshape(_ROWS2D, 128)
    col_p = jnp.concatenate([col, padidx]).reshape(_ROWS2D, 128)
    return row_p, col_p


def _lightgcn_sc(emb0, row2d, col2d, deg, K):
    dinv = jnp.where(deg > 0, lax.rsqrt(jnp.maximum(deg, 1.0)), 0.0)
    x0 = jnp.pad(emb0, ((0, _NPAD - _N), (0, 0)))
    xs = [x0.reshape(_NPAD, _NSL, _SLD)[:, i, :] for i in range(_NSL)]
    sums = list(xs)
    zs = [x * dinv[:, None] for x in xs]
    for _ in range(K):
        os = _prop_kernel(*zs, row2d, col2d)
        xs = [o * dinv[:, None] for o in os]
        sums = [a + b for a, b in zip(sums, xs)]
        zs = [x * dinv[:, None] for x in xs]
    mean = jnp.stack(sums, axis=1).reshape(_NPAD, 128) / (K + 1)
    return mean[:_N]


def kernel(users_emb, ui_items_emb, baskets_emb, bi_items_emb,
           u2i_src, u2i_dst, b2i_src, b2i_dst):
    row1 = jnp.concatenate([u2i_src, u2i_dst + _NU])
    col1 = jnp.concatenate([u2i_dst + _NU, u2i_src])
    row2 = jnp.concatenate([b2i_src, b2i_dst + _NB])
    col2 = jnp.concatenate([b2i_dst + _NB, b2i_src])
    r1, c1 = _pad_edges(row1, col1)
    r2, c2 = _pad_edges(row2, col2)

    deg1, deg2 = _deg_kernel(r1, r2)

    emb0_1 = jnp.concatenate([users_emb, ui_items_emb], axis=0)
    emb0_2 = jnp.concatenate([baskets_emb, bi_items_emb], axis=0)
    final1 = _lightgcn_sc(emb0_1, r1, c1, deg1, 2)
    final2 = _lightgcn_sc(emb0_2, r2, c2, deg2, 3)
    return (final1[:_NU], final1[_NU:], final2[:_NB], final2[_NB:])


# speedup vs baseline: 1.4493x; 1.4493x over previous
import jax
import jax.numpy as jnp
from jax import lax
from jax.experimental import pallas as pl
from jax.experimental.pallas import tpu as pltpu
from jax.experimental.pallas import tpu_sc as plsc

_NU, _NI, _NB = 20000, 30000, 20000
_N = _NU + _NI
_NPAD = 50176
_NODES_PER_TILE = _NPAD // 16
_E = 800000
_CHUNK = 256
_IDXR = _CHUNK // 128
_CHUNKS_PER_TILE = 196
_EDGES_PER_TILE = _CHUNK * _CHUNKS_PER_TILE
_EPAD = 16 * _EDGES_PER_TILE
_ROWS2D = _EPAD // 128
_ROWS_PER_TILE = _ROWS2D // 16

_mesh = plsc.VectorSubcoreMesh(core_axis_name="c", subcore_axis_name="s")
_cparams = pltpu.CompilerParams(use_tc_tiling_on_sc=False)


def _deg_body(c1, c2, d1, d2, dacc, idx, ones, zeros1d, sem):
    core = lax.axis_index("c")
    sub = lax.axis_index("s")
    node_lo = sub * _NODES_PER_TILE
    comb_base = 2 * sub * _ROWS_PER_TILE

    def fill_ones(i, _):
        ones[pl.ds(i * 16, 16)] = jnp.full((16,), 1.0, jnp.float32)
        return _
    lax.fori_loop(0, 8, fill_ones, 0)

    def zfill(i, _):
        zeros1d[pl.ds(i * 16, 16)] = jnp.zeros((16,), jnp.float32)
        return _
    lax.fori_loop(0, _NODES_PER_TILE // 16, zfill, 0)

    for g, (cref, dref) in enumerate(((c1, d1), (c2, d2))):
        @pl.when(core == g)
        def _():
            pltpu.sync_copy(zeros1d, dacc.at[pl.ds(node_lo, _NODES_PER_TILE)])
            plsc.subcore_barrier()

            def chunk(g_, _):
                boff = comb_base + g_ * 8
                pltpu.sync_copy(cref.at[pl.ds(boff, 8)], idx)
                for j in range(4):
                    pltpu.sync_copy(ones.at[pl.ds(0, 128)],
                                    dacc.at[idx.at[2 * j + 1]], add=True)
                return _
            lax.fori_loop(0, (2 * _ROWS_PER_TILE) // 8, chunk, 0)
            plsc.subcore_barrier()
            pltpu.sync_copy(dacc.at[pl.ds(node_lo, _NODES_PER_TILE)],
                            dref.at[pl.ds(node_lo, _NODES_PER_TILE)])


_deg_kernel = pl.kernel(
    _deg_body,
    out_type=[jax.ShapeDtypeStruct((_NPAD,), jnp.float32)] * 2,
    mesh=_mesh,
    scratch_types=[
        pltpu.VMEM_SHARED((_NPAD,), jnp.float32),
        pltpu.VMEM((8, 128), jnp.int32),
        pltpu.VMEM((128,), jnp.float32),
        pltpu.VMEM((_NODES_PER_TILE,), jnp.float32),
        pltpu.SemaphoreType.DMA,
    ],
    compiler_params=_cparams,
)


def _prop_body(z0, z1, z2, z3, comb, o0, o1, o2, o3,
               acc, idxc, rows, isem, gsem, ssem):
    core = lax.axis_index("c")
    sub = lax.axis_index("s")
    node_lo = sub * _NODES_PER_TILE
    comb_base = 2 * sub * _ROWS_PER_TILE

    for s, (zref, oref) in enumerate(((z0, o0), (z1, o1), (z2, o2), (z3, o3))):
        @pl.when(core == s // 2)
        def _():
            def zb(i, _):
                rows[0, i, pl.ds(0, 16)] = jnp.zeros((16,), jnp.float32)
                rows[0, i, pl.ds(16, 16)] = jnp.zeros((16,), jnp.float32)
                return _
            lax.fori_loop(0, _CHUNK, zb, 0)
            def zcopy(i, _):
                pltpu.sync_copy(rows.at[0],
                                acc.at[pl.ds(node_lo + i * _CHUNK, _CHUNK)])
                return _
            lax.fori_loop(0, 12, zcopy, 0)
            pltpu.sync_copy(rows.at[0].at[pl.ds(0, 64)],
                            acc.at[pl.ds(node_lo + 12 * _CHUNK, 64)])
            plsc.subcore_barrier()

            def idx_cp(g, p):
                boff = comb_base + 4 * g
                return pltpu.make_async_copy(comb.at[pl.ds(boff, 4)],
                                             idxc.at[p], isem)

            def gathers(p):
                return [pltpu.make_async_copy(
                            zref.at[idxc.at[p].at[2 * j]],
                            rows.at[p].at[pl.ds(j * 128, 128)], gsem)
                        for j in range(_IDXR)]

            def scat_chunk(p):
                descs = [pltpu.async_copy(
                             rows.at[p].at[pl.ds(j * 128, 128)],
                             acc.at[idxc.at[p].at[2 * j + 1]], ssem, add=True)
                         for j in range(_IDXR)]
                for d in descs:
                    d.wait()

            def body(g, b, issue_idx):
                p2 = (b + 2) % 3
                for d in gathers(b):
                    d.wait()
                idx_cp(g + 2, p2).wait()
                for d in gathers(p2):
                    d.start()
                scat_chunk(b)
                if issue_idx:
                    idx_cp(g + 3, b).start()

            idx_cp(0, 0).start()
            idx_cp(0, 0).wait()
            for d in gathers(0):
                d.start()
            idx_cp(1, 1).start()
            idx_cp(1, 1).wait()
            for d in gathers(1):
                d.start()
            idx_cp(2, 2).start()

            def triple(gj, _):
                for b in range(3):
                    body(3 * gj + b, b, True)
                return _
            lax.fori_loop(0, (_CHUNKS_PER_TILE - 4) // 3, triple, 0)

            gl = _CHUNKS_PER_TILE - 4
            body(gl, 0, True)
            for d in gathers(1):
                d.wait()
            idx_cp(gl + 3, 0).wait()
            for d in gathers(0):
                d.start()
            scat_chunk(1)
            for d in gathers(2):
                d.wait()
            scat_chunk(2)
            for d in gathers(0):
                d.wait()
            scat_chunk(0)

            plsc.subcore_barrier()
            pltpu.sync_copy(acc.at[pl.ds(node_lo, _NODES_PER_TILE)],
                            oref.at[pl.ds(node_lo, _NODES_PER_TILE)])
            plsc.subcore_barrier()


_prop_kernel = pl.kernel(
    _prop_body,
    out_type=[jax.ShapeDtypeStruct((_NPAD, 32), jnp.float32)] * 4,
    mesh=_mesh,
    scratch_types=[
        pltpu.VMEM_SHARED((_NPAD, 32), jnp.float32),
        pltpu.VMEM((3, 4, 128), jnp.int32),
        pltpu.VMEM((3, _CHUNK, 32), jnp.float32),
        pltpu.SemaphoreType.DMA,
        pltpu.SemaphoreType.DMA,
        pltpu.SemaphoreType.DMA,
    ],
    compiler_params=_cparams,
)


def _pad_and_interleave(row, col):
    pad = _EPAD - _E
    padidx = _N + (jnp.arange(pad, dtype=jnp.int32) % 16)
    row_p = jnp.concatenate([row, padidx]).reshape(_ROWS2D, 1, 128)
    col_p = jnp.concatenate([col, padidx]).reshape(_ROWS2D, 1, 128)
    return jnp.concatenate([col_p, row_p], axis=1).reshape(2 * _ROWS2D, 128)


def _lightgcn_sc(emb0, comb, deg, K):
    dinv = jnp.where(deg > 0, lax.rsqrt(jnp.maximum(deg, 1.0)), 0.0)
    x0 = jnp.pad(emb0, ((0, _NPAD - _N), (0, 0)))
    xs = [x0.reshape(_NPAD, 4, 32)[:, i, :] for i in range(4)]
    sums = list(xs)
    zs = [x * dinv[:, None] for x in xs]
    for _ in range(K):
        os = _prop_kernel(zs[0], zs[1], zs[2], zs[3], comb)
        xs = [o * dinv[:, None] for o in os]
        sums = [a + b for a, b in zip(sums, xs)]
        zs = [x * dinv[:, None] for x in xs]
    mean = jnp.stack(sums, axis=1).reshape(_NPAD, 128) / (K + 1)
    return mean[:_N]


def kernel(users_emb, ui_items_emb, baskets_emb, bi_items_emb,
           u2i_src, u2i_dst, b2i_src, b2i_dst):
    row1 = jnp.concatenate([u2i_src, u2i_dst + _NU])
    col1 = jnp.concatenate([u2i_dst + _NU, u2i_src])
    row2 = jnp.concatenate([b2i_src, b2i_dst + _NB])
    col2 = jnp.concatenate([b2i_dst + _NB, b2i_src])
    comb1 = _pad_and_interleave(row1, col1)
    comb2 = _pad_and_interleave(row2, col2)

    deg1, deg2 = _deg_kernel(comb1, comb2)

    emb0_1 = jnp.concatenate([users_emb, ui_items_emb], axis=0)
    emb0_2 = jnp.concatenate([baskets_emb, bi_items_emb], axis=0)
    final1 = _lightgcn_sc(emb0_1, comb1, deg1, 2)
    final2 = _lightgcn_sc(emb0_2, comb2, deg2, 3)
    return (final1[:_NU], final1[_NU:], final2[:_NB], final2[_NB:])
